# baseline (speedup 1.0000x reference)
import functools

import jax
import jax.numpy as jnp
from jax.experimental import pallas as pl
from jax.experimental.pallas import tpu as pltpu

NUM_EXPERTS = 8
TOP_K = 2
D_IN = 1024
D_HID = 4096
D_OUT = 1024
N_TOK = 2048

BLK = 256
GMAX = 24
PPAD = GMAX * BLK
NH = 4
HB = D_HID // NH


def _gating(x, w_gate):
    logits = x @ w_gate
    p = jax.nn.softmax(logits, axis=1)
    lane = jnp.arange(NUM_EXPERTS)[None, :]
    m1 = jnp.max(p, 1, keepdims=True)
    i1 = jnp.min(jnp.where(p == m1, lane, NUM_EXPERTS), 1, keepdims=True)
    p2 = jnp.where(lane == i1, -jnp.inf, p)
    m2 = jnp.max(p2, 1, keepdims=True)
    i2 = jnp.min(jnp.where(p2 == m2, lane, NUM_EXPERTS), 1, keepdims=True)
    denom = m1 + m2 + 1e-6
    g1, g2 = m1 / denom, m2 / denom
    oh1 = (lane == i1).astype(jnp.float32)
    oh2 = (lane == i2).astype(jnp.float32)
    importance = (oh1 * g1 + oh2 * g2).sum(0)
    load = (oh1 + oh2).sum(0)

    def cv2(v):
        mu = v.mean()
        var = ((v - mu) ** 2).sum() / (NUM_EXPERTS - 1)
        return var / (mu * mu + 1e-10)

    loss = 1e-2 * (cv2(importance) + cv2(load))
    return i1[:, 0], i2[:, 0], g1[:, 0], g2[:, 0], loss


def _route(i1, i2, g1, g2):
    pe = jnp.stack([i1, i2], 1).reshape(-1)
    pg = jnp.stack([g1, g2], 1).reshape(-1)
    pt = jnp.repeat(jnp.arange(N_TOK), TOP_K)
    onehot = (pe[:, None] == jnp.arange(NUM_EXPERTS)[None, :]).astype(jnp.int32)
    counts = onehot.sum(0)
    rank = jnp.cumsum(onehot, 0) - 1
    rank = jnp.take_along_axis(rank, pe[:, None], 1)[:, 0]
    nb = (counts + BLK - 1) // BLK
    cum_nb = jnp.cumsum(nb)
    bstart = cum_nb - nb
    dest = bstart[pe] * BLK + rank
    sorted_tid = jnp.zeros(PPAD, jnp.int32).at[dest].set(pt.astype(jnp.int32))
    sorted_gate = jnp.zeros(PPAD, jnp.float32).at[dest].set(pg)
    pos = dest.reshape(N_TOK, TOP_K)
    g_used = cum_nb[NUM_EXPERTS - 1]
    bidx = jnp.arange(GMAX)
    be = jnp.searchsorted(cum_nb, bidx, side="right")
    be_last = jnp.searchsorted(cum_nb, g_used - 1, side="right")
    bv = (bidx < g_used).astype(jnp.int32)
    be = jnp.where(bv == 1, be, be_last).astype(jnp.int32)
    return sorted_tid, sorted_gate, be, bv, pos


def _ffn_body(be_ref, bv_ref, gate_ref, xb_ref, w1_ref, b1_ref, w2_ref,
              b2_ref, out_ref, acc_ref):
    g = pl.program_id(0)
    h = pl.program_id(1)

    @pl.when(bv_ref[g] == 1)
    def _():
        xb = xb_ref[...]
        hb = jnp.dot(xb, w1_ref[0], preferred_element_type=jnp.float32)
        hb = jnp.maximum(hb + b1_ref[0], 0.0).astype(jnp.bfloat16)
        contrib = jnp.dot(hb, w2_ref[0], preferred_element_type=jnp.float32)

        @pl.when(h == 0)
        def _():
            acc_ref[...] = contrib

        @pl.when(h > 0)
        def _():
            acc_ref[...] += contrib

        @pl.when(h == NH - 1)
        def _():
            logits = acc_ref[...] + b2_ref[0]
            m = jnp.max(logits, axis=1, keepdims=True)
            ex = jnp.exp(logits - m)
            o = ex / jnp.sum(ex, axis=1, keepdims=True)
            out_ref[...] = o * gate_ref[0, 0][:, None]


@functools.partial(jax.jit, static_argnames=())
def _ffn(x_sorted, gate3d, W1, b1, W2, b2, be, bv):
    grid_spec = pltpu.PrefetchScalarGridSpec(
        num_scalar_prefetch=2,
        grid=(GMAX, NH),
        in_specs=[
            pl.BlockSpec((1, 1, BLK), lambda g, h, be, bv: (g, 0, 0)),
            pl.BlockSpec((BLK, D_IN), lambda g, h, be, bv: (g, 0)),
            pl.BlockSpec((1, D_IN, HB), lambda g, h, be, bv: (be[g], 0, h)),
            pl.BlockSpec((1, 1, HB), lambda g, h, be, bv: (be[g] * NH + h, 0, 0)),
            pl.BlockSpec((1, HB, D_OUT), lambda g, h, be, bv: (be[g], h, 0)),
            pl.BlockSpec((1, 1, D_OUT), lambda g, h, be, bv: (be[g], 0, 0)),
        ],
        out_specs=pl.BlockSpec((BLK, D_OUT), lambda g, h, be, bv: (g, 0)),
        scratch_shapes=[pltpu.VMEM((BLK, D_OUT), jnp.float32)],
    )
    return pl.pallas_call(
        _ffn_body,
        grid_spec=grid_spec,
        out_shape=jax.ShapeDtypeStruct((PPAD, D_OUT), jnp.float32),
        compiler_params=pltpu.CompilerParams(
            dimension_semantics=("arbitrary", "arbitrary")),
    )(be, bv, gate3d, x_sorted, W1,
      b1.reshape(NUM_EXPERTS * NH, 1, HB), W2,
      b2.reshape(NUM_EXPERTS, 1, D_OUT))


def kernel(x, w_gate, w_noise, W1, b1, W2, b2):
    i1, i2, g1, g2, loss = _gating(x, w_gate)
    sorted_tid, sorted_gate, be, bv, pos = _route(i1, i2, g1, g2)
    x_sorted = x[sorted_tid]
    gate3d = sorted_gate.reshape(GMAX, 1, BLK)
    o_sorted = _ffn(x_sorted.astype(jnp.bfloat16), gate3d,
                    W1.astype(jnp.bfloat16), b1,
                    W2.astype(jnp.bfloat16), b2, be, bv)
    y = o_sorted[pos[:, 0]] + o_sorted[pos[:, 1]]
    return y, loss


# speedup vs baseline: 1.1419x; 1.1419x over previous
import functools

import jax
import jax.numpy as jnp
from jax.experimental import pallas as pl
from jax.experimental.pallas import tpu as pltpu

NUM_EXPERTS = 8
TOP_K = 2
D_IN = 1024
D_HID = 4096
D_OUT = 1024
N_TOK = 2048

BLK = 1024
GMAX = 11
PPAD = GMAX * BLK
NH = 4
HB = D_HID // NH


def _gating(x, w_gate):
    logits = x @ w_gate
    p = jax.nn.softmax(logits, axis=1)
    lane = jnp.arange(NUM_EXPERTS)[None, :]
    m1 = jnp.max(p, 1, keepdims=True)
    i1 = jnp.min(jnp.where(p == m1, lane, NUM_EXPERTS), 1, keepdims=True)
    p2 = jnp.where(lane == i1, -jnp.inf, p)
    m2 = jnp.max(p2, 1, keepdims=True)
    i2 = jnp.min(jnp.where(p2 == m2, lane, NUM_EXPERTS), 1, keepdims=True)
    denom = m1 + m2 + 1e-6
    g1, g2 = m1 / denom, m2 / denom
    oh1 = (lane == i1).astype(jnp.float32)
    oh2 = (lane == i2).astype(jnp.float32)
    importance = (oh1 * g1 + oh2 * g2).sum(0)
    load = (oh1 + oh2).sum(0)

    def cv2(v):
        mu = v.mean()
        var = ((v - mu) ** 2).sum() / (NUM_EXPERTS - 1)
        return var / (mu * mu + 1e-10)

    loss = 1e-2 * (cv2(importance) + cv2(load))
    return i1[:, 0], i2[:, 0], g1[:, 0], g2[:, 0], loss


def _route(i1, i2, g1, g2):
    pe = jnp.stack([i1, i2], 1).reshape(-1)
    pg = jnp.stack([g1, g2], 1).reshape(-1)
    pt = jnp.repeat(jnp.arange(N_TOK), TOP_K)
    onehot = (pe[:, None] == jnp.arange(NUM_EXPERTS)[None, :]).astype(jnp.int32)
    counts = onehot.sum(0)
    rank = jnp.cumsum(onehot, 0) - 1
    rank = jnp.take_along_axis(rank, pe[:, None], 1)[:, 0]
    nb = (counts + BLK - 1) // BLK
    cum_nb = jnp.cumsum(nb)
    bstart = cum_nb - nb
    dest = bstart[pe] * BLK + rank
    sorted_tid = jnp.zeros(PPAD, jnp.int32).at[dest].set(pt.astype(jnp.int32))
    sorted_gate = jnp.zeros(PPAD, jnp.float32).at[dest].set(pg)
    pos = dest.reshape(N_TOK, TOP_K)
    g_used = cum_nb[NUM_EXPERTS - 1]
    bidx = jnp.arange(GMAX)
    be = jnp.searchsorted(cum_nb, bidx, side="right")
    be_last = jnp.searchsorted(cum_nb, g_used - 1, side="right")
    bv = (bidx < g_used).astype(jnp.int32)
    be = jnp.where(bv == 1, be, be_last).astype(jnp.int32)
    return sorted_tid, sorted_gate, be, bv, pos


def _ffn_body(be_ref, bv_ref, gate_ref, xb_ref, w1_ref, b1_ref, w2_ref,
              b2_ref, out_ref, acc_ref):
    g = pl.program_id(0)
    h = pl.program_id(1)

    @pl.when(bv_ref[g] == 1)
    def _():
        xb = xb_ref[...].astype(jnp.bfloat16)
        hb = jnp.dot(xb, w1_ref[0].astype(jnp.bfloat16),
                     preferred_element_type=jnp.float32)
        hb = jnp.maximum(hb + b1_ref[0], 0.0).astype(jnp.bfloat16)
        contrib = jnp.dot(hb, w2_ref[0].astype(jnp.bfloat16),
                          preferred_element_type=jnp.float32)

        @pl.when(h == 0)
        def _():
            acc_ref[...] = contrib

        @pl.when(h > 0)
        def _():
            acc_ref[...] += contrib

        @pl.when(h == NH - 1)
        def _():
            logits = acc_ref[...] + b2_ref[0]
            m = jnp.max(logits, axis=1, keepdims=True)
            ex = jnp.exp(logits - m)
            o = ex / jnp.sum(ex, axis=1, keepdims=True)
            out_ref[...] = o * gate_ref[0, 0][:, None]


@functools.partial(jax.jit, static_argnames=())
def _ffn(x_sorted, gate3d, W1, b1, W2, b2, be, bv):
    grid_spec = pltpu.PrefetchScalarGridSpec(
        num_scalar_prefetch=2,
        grid=(GMAX, NH),
        in_specs=[
            pl.BlockSpec((1, 1, BLK), lambda g, h, be, bv: (g, 0, 0)),
            pl.BlockSpec((BLK, D_IN), lambda g, h, be, bv: (g, 0)),
            pl.BlockSpec((1, D_IN, HB), lambda g, h, be, bv: (be[g], 0, h)),
            pl.BlockSpec((1, 1, HB), lambda g, h, be, bv: (be[g] * NH + h, 0, 0)),
            pl.BlockSpec((1, HB, D_OUT), lambda g, h, be, bv: (be[g], h, 0)),
            pl.BlockSpec((1, 1, D_OUT), lambda g, h, be, bv: (be[g], 0, 0)),
        ],
        out_specs=pl.BlockSpec((BLK, D_OUT), lambda g, h, be, bv: (g, 0)),
        scratch_shapes=[pltpu.VMEM((BLK, D_OUT), jnp.float32)],
    )
    return pl.pallas_call(
        _ffn_body,
        grid_spec=grid_spec,
        out_shape=jax.ShapeDtypeStruct((PPAD, D_OUT), jnp.float32),
        compiler_params=pltpu.CompilerParams(
            dimension_semantics=("arbitrary", "arbitrary")),
    )(be, bv, gate3d, x_sorted, W1,
      b1.reshape(NUM_EXPERTS * NH, 1, HB), W2,
      b2.reshape(NUM_EXPERTS, 1, D_OUT))


def kernel(x, w_gate, w_noise, W1, b1, W2, b2):
    i1, i2, g1, g2, loss = _gating(x, w_gate)
    sorted_tid, sorted_gate, be, bv, pos = _route(i1, i2, g1, g2)
    x_sorted = x[sorted_tid]
    gate3d = sorted_gate.reshape(GMAX, 1, BLK)
    o_sorted = _ffn(x_sorted, gate3d, W1, b1, W2, b2, be, bv)
    y = o_sorted[pos[:, 0]] + o_sorted[pos[:, 1]]
    return y, loss


# speedup vs baseline: 1.4007x; 1.2266x over previous
import functools

import jax
import jax.numpy as jnp
from jax.experimental import pallas as pl
from jax.experimental.pallas import tpu as pltpu

NUM_EXPERTS = 8
TOP_K = 2
D_IN = 1024
D_HID = 4096
D_OUT = 1024
N_TOK = 2048

BLK = 512
GMAX = 15
PPAD = GMAX * BLK
NH = 2
HB = D_HID // NH


def _gating(x, w_gate):
    logits = x @ w_gate
    p = jax.nn.softmax(logits, axis=1)
    lane = jnp.arange(NUM_EXPERTS)[None, :]
    m1 = jnp.max(p, 1, keepdims=True)
    i1 = jnp.min(jnp.where(p == m1, lane, NUM_EXPERTS), 1, keepdims=True)
    p2 = jnp.where(lane == i1, -jnp.inf, p)
    m2 = jnp.max(p2, 1, keepdims=True)
    i2 = jnp.min(jnp.where(p2 == m2, lane, NUM_EXPERTS), 1, keepdims=True)
    denom = m1 + m2 + 1e-6
    g1, g2 = m1 / denom, m2 / denom
    oh1 = (lane == i1).astype(jnp.float32)
    oh2 = (lane == i2).astype(jnp.float32)
    importance = (oh1 * g1 + oh2 * g2).sum(0)
    load = (oh1 + oh2).sum(0)

    def cv2(v):
        mu = v.mean()
        var = ((v - mu) ** 2).sum() / (NUM_EXPERTS - 1)
        return var / (mu * mu + 1e-10)

    loss = 1e-2 * (cv2(importance) + cv2(load))
    return i1[:, 0], i2[:, 0], g1[:, 0], g2[:, 0], loss


def _route(i1, i2, g1, g2):
    pe = jnp.stack([i1, i2], 1).reshape(-1)
    pg = jnp.stack([g1, g2], 1).reshape(-1)
    pt = jnp.repeat(jnp.arange(N_TOK), TOP_K)
    onehot = (pe[:, None] == jnp.arange(NUM_EXPERTS)[None, :]).astype(jnp.int32)
    counts = onehot.sum(0)
    rank = jnp.cumsum(onehot, 0) - 1
    rank = jnp.take_along_axis(rank, pe[:, None], 1)[:, 0]
    nb = (counts + BLK - 1) // BLK
    cum_nb = jnp.cumsum(nb)
    bstart = cum_nb - nb
    dest = bstart[pe] * BLK + rank
    sorted_tid = jnp.zeros(PPAD, jnp.int32).at[dest].set(pt.astype(jnp.int32))
    sorted_gate = jnp.zeros(PPAD, jnp.float32).at[dest].set(pg)
    pos = dest.reshape(N_TOK, TOP_K)
    g_used = cum_nb[NUM_EXPERTS - 1]
    bidx = jnp.arange(GMAX)
    be = jnp.searchsorted(cum_nb, bidx, side="right")
    be_last = jnp.searchsorted(cum_nb, g_used - 1, side="right")
    bv = (bidx < g_used).astype(jnp.int32)
    be = jnp.where(bv == 1, be, be_last).astype(jnp.int32)
    return sorted_tid, sorted_gate, be, bv, pos


def _ffn_body(be_ref, bv_ref, gate_ref, xb_ref, w1_ref, b1_ref, w2_ref,
              b2_ref, out_ref, acc_ref, xbf_ref):
    g = pl.program_id(0)
    h = pl.program_id(1)

    @pl.when(bv_ref[g] == 1)
    def _():
        @pl.when(h == 0)
        def _():
            xbf_ref[...] = xb_ref[...].astype(jnp.bfloat16)

        hb = jnp.dot(xbf_ref[...], w1_ref[0].astype(jnp.bfloat16),
                     preferred_element_type=jnp.float32)
        hb = jnp.maximum(hb + b1_ref[0], 0.0).astype(jnp.bfloat16)
        contrib = jnp.dot(hb, w2_ref[0].astype(jnp.bfloat16),
                          preferred_element_type=jnp.float32)

        @pl.when(h == 0)
        def _():
            acc_ref[...] = contrib

        @pl.when(h > 0)
        def _():
            acc_ref[...] += contrib

        @pl.when(h == NH - 1)
        def _():
            logits = acc_ref[...] + b2_ref[0]
            m = jnp.max(logits, axis=1, keepdims=True)
            ex = jnp.exp(logits - m)
            o = ex / jnp.sum(ex, axis=1, keepdims=True)
            out_ref[...] = o * gate_ref[0, 0][:, None]


@functools.partial(jax.jit, static_argnames=())
def _ffn(x_sorted, gate3d, W1, b1, W2, b2, be, bv):
    grid_spec = pltpu.PrefetchScalarGridSpec(
        num_scalar_prefetch=2,
        grid=(GMAX, NH),
        in_specs=[
            pl.BlockSpec((1, 1, BLK), lambda g, h, be, bv: (g, 0, 0)),
            pl.BlockSpec((BLK, D_IN), lambda g, h, be, bv: (g, 0)),
            pl.BlockSpec((1, D_IN, HB), lambda g, h, be, bv: (be[g], 0, h)),
            pl.BlockSpec((1, 1, HB), lambda g, h, be, bv: (be[g] * NH + h, 0, 0)),
            pl.BlockSpec((1, HB, D_OUT), lambda g, h, be, bv: (be[g], h, 0)),
            pl.BlockSpec((1, 1, D_OUT), lambda g, h, be, bv: (be[g], 0, 0)),
        ],
        out_specs=pl.BlockSpec((BLK, D_OUT), lambda g, h, be, bv: (g, 0)),
        scratch_shapes=[pltpu.VMEM((BLK, D_OUT), jnp.float32),
                        pltpu.VMEM((BLK, D_IN), jnp.bfloat16)],
    )
    return pl.pallas_call(
        _ffn_body,
        grid_spec=grid_spec,
        out_shape=jax.ShapeDtypeStruct((PPAD, D_OUT), jnp.float32),
        compiler_params=pltpu.CompilerParams(
            dimension_semantics=("arbitrary", "arbitrary")),
    )(be, bv, gate3d, x_sorted, W1,
      b1.reshape(NUM_EXPERTS * NH, 1, HB), W2,
      b2.reshape(NUM_EXPERTS, 1, D_OUT))


def kernel(x, w_gate, w_noise, W1, b1, W2, b2):
    i1, i2, g1, g2, loss = _gating(x, w_gate)
    sorted_tid, sorted_gate, be, bv, pos = _route(i1, i2, g1, g2)
    x_sorted = x[sorted_tid]
    gate3d = sorted_gate.reshape(GMAX, 1, BLK)
    o_sorted = _ffn(x_sorted, gate3d, W1, b1, W2, b2, be, bv)
    y = o_sorted[pos[:, 0]] + o_sorted[pos[:, 1]]
    return y, loss


# speedup vs baseline: 1.7897x; 1.2777x over previous
import functools

import jax
import jax.numpy as jnp
from jax import lax
from jax.experimental import pallas as pl
from jax.experimental.pallas import tpu as pltpu
from jax.experimental.pallas import tpu_sc as plsc

NUM_EXPERTS = 8
TOP_K = 2
D_IN = 1024
D_HID = 4096
D_OUT = 1024
N_TOK = 2048

BLK = 512
GMAX = 15
PPAD = GMAX * BLK
NH = 2
HB = D_HID // NH

NW = 32
TPW = N_TOK // NW
CC = 32


def _gating(x, w_gate):
    logits = x @ w_gate
    p = jax.nn.softmax(logits, axis=1)
    lane = jnp.arange(NUM_EXPERTS)[None, :]
    m1 = jnp.max(p, 1, keepdims=True)
    i1 = jnp.min(jnp.where(p == m1, lane, NUM_EXPERTS), 1, keepdims=True)
    p2 = jnp.where(lane == i1, -jnp.inf, p)
    m2 = jnp.max(p2, 1, keepdims=True)
    i2 = jnp.min(jnp.where(p2 == m2, lane, NUM_EXPERTS), 1, keepdims=True)
    denom = m1 + m2 + 1e-6
    g1, g2 = m1 / denom, m2 / denom
    oh1 = (lane == i1).astype(jnp.float32)
    oh2 = (lane == i2).astype(jnp.float32)
    importance = (oh1 * g1 + oh2 * g2).sum(0)
    load = (oh1 + oh2).sum(0)

    def cv2(v):
        mu = v.mean()
        var = ((v - mu) ** 2).sum() / (NUM_EXPERTS - 1)
        return var / (mu * mu + 1e-10)

    loss = 1e-2 * (cv2(importance) + cv2(load))
    return i1[:, 0], i2[:, 0], g1[:, 0], g2[:, 0], loss


def _route(i1, i2, g1, g2):
    pe = jnp.stack([i1, i2], 1).reshape(-1)
    onehot = (pe[:, None] == jnp.arange(NUM_EXPERTS)[None, :]).astype(jnp.int32)
    counts = onehot.sum(0)
    rank = jnp.cumsum(onehot, 0) - 1
    rank = jnp.take_along_axis(rank, pe[:, None], 1)[:, 0]
    nb = (counts + BLK - 1) // BLK
    cum_nb = jnp.cumsum(nb)
    bstart = cum_nb - nb
    dest = (bstart[pe] * BLK + rank).reshape(N_TOK, TOP_K).astype(jnp.int32)
    g_used = cum_nb[NUM_EXPERTS - 1]
    bidx = jnp.arange(GMAX)
    be = jnp.searchsorted(cum_nb, bidx, side="right")
    be_last = jnp.searchsorted(cum_nb, g_used - 1, side="right")
    bv = (bidx < g_used).astype(jnp.int32)
    be = jnp.where(bv == 1, be, be_last).astype(jnp.int32)
    return dest, be, bv


def _dispatch_body(x_hbm, d0_hbm, d1_hbm, xs_hbm, xv, i0v, i1v, sem):
    wid = lax.axis_index("s") * 2 + lax.axis_index("c")
    base = wid * TPW
    pltpu.sync_copy(x_hbm.at[pl.ds(base, TPW)], xv)
    pltpu.sync_copy(d0_hbm.at[pl.ds(base, TPW)], i0v)
    pltpu.sync_copy(d1_hbm.at[pl.ds(base, TPW)], i1v)
    pltpu.async_copy(xv, xs_hbm.at[i0v], sem).wait()
    pltpu.async_copy(xv, xs_hbm.at[i1v], sem).wait()


def _dispatch(x, d0, d1):
    return pl.kernel(
        _dispatch_body,
        out_type=jax.ShapeDtypeStruct((PPAD, D_IN), jnp.float32),
        mesh=plsc.VectorSubcoreMesh(core_axis_name="c", subcore_axis_name="s"),
        scratch_types=[pltpu.VMEM((TPW, D_IN), jnp.float32),
                       pltpu.VMEM((TPW,), jnp.int32),
                       pltpu.VMEM((TPW,), jnp.int32),
                       pltpu.SemaphoreType.DMA],
    )(x, d0, d1)


def _combine_body(o_hbm, d0_hbm, d1_hbm, g0_hbm, g1_hbm, y_hbm,
                  i0v, i1v, g0v, g1v, b0, b1, yv, sem):
    wid = lax.axis_index("s") * 2 + lax.axis_index("c")
    for c in range(TPW // CC):
        base = wid * TPW + c * CC
        pltpu.sync_copy(d0_hbm.at[pl.ds(base, CC)], i0v)
        pltpu.sync_copy(d1_hbm.at[pl.ds(base, CC)], i1v)
        pltpu.sync_copy(g0_hbm.at[pl.ds(base, CC)], g0v)
        pltpu.sync_copy(g1_hbm.at[pl.ds(base, CC)], g1v)
        pltpu.async_copy(o_hbm.at[i0v], b0, sem).wait()
        pltpu.async_copy(o_hbm.at[i1v], b1, sem).wait()

        def tok(i, carry):
            ga = g0v[i, :]
            gb = g1v[i, :]
            for j in range(D_OUT // 16):
                sl = pl.ds(j * 16, 16)
                yv[i, sl] = ga * b0[i, sl] + gb * b1[i, sl]
            return carry

        lax.fori_loop(0, CC, tok, 0)
        pltpu.sync_copy(yv, y_hbm.at[pl.ds(base, CC)])


def _combine(o_sorted, d0, d1, g0, g1):
    return pl.kernel(
        _combine_body,
        out_type=jax.ShapeDtypeStruct((N_TOK, D_OUT), jnp.float32),
        mesh=plsc.VectorSubcoreMesh(core_axis_name="c", subcore_axis_name="s"),
        scratch_types=[pltpu.VMEM((CC,), jnp.int32),
                       pltpu.VMEM((CC,), jnp.int32),
                       pltpu.VMEM((CC, 16), jnp.float32),
                       pltpu.VMEM((CC, 16), jnp.float32),
                       pltpu.VMEM((CC, D_OUT), jnp.float32),
                       pltpu.VMEM((CC, D_OUT), jnp.float32),
                       pltpu.VMEM((CC, D_OUT), jnp.float32),
                       pltpu.SemaphoreType.DMA],
    )(o_sorted, d0, d1, g0, g1)


def _ffn_body(be_ref, bv_ref, xb_ref, w1_ref, b1_ref, w2_ref,
              b2_ref, out_ref, acc_ref, xbf_ref):
    g = pl.program_id(0)
    h = pl.program_id(1)

    @pl.when(bv_ref[g] == 1)
    def _():
        @pl.when(h == 0)
        def _():
            xbf_ref[...] = xb_ref[...].astype(jnp.bfloat16)

        hb = jnp.dot(xbf_ref[...], w1_ref[0].astype(jnp.bfloat16),
                     preferred_element_type=jnp.float32)
        hb = jnp.maximum(hb + b1_ref[0], 0.0).astype(jnp.bfloat16)
        contrib = jnp.dot(hb, w2_ref[0].astype(jnp.bfloat16),
                          preferred_element_type=jnp.float32)

        @pl.when(h == 0)
        def _():
            acc_ref[...] = contrib

        @pl.when(h > 0)
        def _():
            acc_ref[...] += contrib

        @pl.when(h == NH - 1)
        def _():
            logits = acc_ref[...] + b2_ref[0]
            m = jnp.max(logits, axis=1, keepdims=True)
            ex = jnp.exp(logits - m)
            out_ref[...] = ex / jnp.sum(ex, axis=1, keepdims=True)


@functools.partial(jax.jit, static_argnames=())
def _ffn(x_sorted, W1, b1, W2, b2, be, bv):
    grid_spec = pltpu.PrefetchScalarGridSpec(
        num_scalar_prefetch=2,
        grid=(GMAX, NH),
        in_specs=[
            pl.BlockSpec((BLK, D_IN), lambda g, h, be, bv: (g, 0)),
            pl.BlockSpec((1, D_IN, HB), lambda g, h, be, bv: (be[g], 0, h)),
            pl.BlockSpec((1, 1, HB), lambda g, h, be, bv: (be[g] * NH + h, 0, 0)),
            pl.BlockSpec((1, HB, D_OUT), lambda g, h, be, bv: (be[g], h, 0)),
            pl.BlockSpec((1, 1, D_OUT), lambda g, h, be, bv: (be[g], 0, 0)),
        ],
        out_specs=pl.BlockSpec((BLK, D_OUT), lambda g, h, be, bv: (g, 0)),
        scratch_shapes=[pltpu.VMEM((BLK, D_OUT), jnp.float32),
                        pltpu.VMEM((BLK, D_IN), jnp.bfloat16)],
    )
    return pl.pallas_call(
        _ffn_body,
        grid_spec=grid_spec,
        out_shape=jax.ShapeDtypeStruct((PPAD, D_OUT), jnp.float32),
        compiler_params=pltpu.CompilerParams(
            dimension_semantics=("arbitrary", "arbitrary")),
    )(be, bv, x_sorted, W1,
      b1.reshape(NUM_EXPERTS * NH, 1, HB), W2,
      b2.reshape(NUM_EXPERTS, 1, D_OUT))


def kernel(x, w_gate, w_noise, W1, b1, W2, b2):
    i1, i2, g1, g2, loss = _gating(x, w_gate)
    dest, be, bv = _route(i1, i2, g1, g2)
    d0 = dest[:, 0]
    d1 = dest[:, 1]
    x_sorted = _dispatch(x, d0, d1)
    o_sorted = _ffn(x_sorted, W1, b1, W2, b2, be, bv)
    g0x = jnp.broadcast_to(g1[:, None], (N_TOK, 16))
    g1x = jnp.broadcast_to(g2[:, None], (N_TOK, 16))
    y = _combine(o_sorted, d0, d1, g0x, g1x)
    return y, loss


# speedup vs baseline: 1.8869x; 1.0543x over previous
import functools

import jax
import jax.numpy as jnp
from jax import lax
from jax.experimental import pallas as pl
from jax.experimental.pallas import tpu as pltpu
from jax.experimental.pallas import tpu_sc as plsc

NUM_EXPERTS = 8
TOP_K = 2
D_IN = 1024
D_HID = 4096
D_OUT = 1024
N_TOK = 2048

BLK = 512
GMAX = 15
PPAD = GMAX * BLK
NH = 2
HB = D_HID // NH

NW = 32
TPW = N_TOK // NW
CC = 32


def _gate_body(x_ref, wg_ref, d0_ref, d1_ref, g0_ref, g1_ref, be_ref,
               bv_ref, loss_ref):
    x = x_ref[...]
    logits = jnp.dot(x, wg_ref[...], preferred_element_type=jnp.float32)
    lane = jax.lax.broadcasted_iota(jnp.int32, (N_TOK, 128), 1)
    valid = lane < NUM_EXPERTS
    lm = jnp.where(valid, logits, -1e30)
    mx = jnp.max(lm, axis=1, keepdims=True)
    ex = jnp.where(valid, jnp.exp(lm - mx), 0.0)
    p = ex / jnp.sum(ex, axis=1, keepdims=True)
    m1 = jnp.max(p, axis=1, keepdims=True)
    i1 = jnp.min(jnp.where(p == m1, lane, 128), axis=1, keepdims=True)
    p2 = jnp.where(lane == i1, -1.0, p)
    m2 = jnp.max(p2, axis=1, keepdims=True)
    i2 = jnp.min(jnp.where(p2 == m2, lane, 128), axis=1, keepdims=True)
    denom = m1 + m2 + 1e-6
    g1 = m1 / denom
    g2 = m2 / denom
    oh1 = (lane == i1)
    oh2 = (lane == i2)
    ohs = oh1.astype(jnp.int32) + oh2.astype(jnp.int32)

    importance = jnp.sum(jnp.where(oh1, g1, 0.0) + jnp.where(oh2, g2, 0.0),
                         axis=0, keepdims=True)
    load = jnp.sum(ohs, axis=0, keepdims=True).astype(jnp.float32)
    mask8 = valid[:1, :]

    def cv2(v):
        mu = jnp.sum(jnp.where(mask8, v, 0.0), axis=1, keepdims=True) / 8.0
        var = jnp.sum(jnp.where(mask8, (v - mu) ** 2, 0.0), axis=1,
                      keepdims=True) / (NUM_EXPERTS - 1)
        return var / (mu * mu + 1e-10)

    loss = 1e-2 * (cv2(importance) + cv2(load))
    loss_ref[...] = loss[:1, :1]

    row = jax.lax.broadcasted_iota(jnp.int32, (N_TOK, 128), 0)
    csum = ohs
    s = 1
    while s < N_TOK:
        csum = csum + jnp.where(row >= s, pltpu.roll(csum, s, 0), 0)
        s *= 2
    cexcl = csum - ohs
    cnt = jnp.sum(ohs, axis=0, keepdims=True)

    nb = (cnt + BLK - 1) // BLK
    cum_nb = nb
    for s in (1, 2, 4):
        cum_nb = cum_nb + jnp.where(lane[:1, :] >= s,
                                    pltpu.roll(cum_nb, s, 1), 0)
    bstart = (cum_nb - nb) * BLK

    rank1 = jnp.sum(jnp.where(oh1, cexcl, 0), axis=1, keepdims=True)
    rank2 = jnp.sum(jnp.where(oh2, cexcl, 0), axis=1, keepdims=True)
    base1 = jnp.sum(jnp.where(oh1, bstart, 0), axis=1, keepdims=True)
    base2 = jnp.sum(jnp.where(oh2, bstart, 0), axis=1, keepdims=True)
    d0_ref[...] = base1 + rank1
    d1_ref[...] = base2 + rank2
    g0_ref[...] = jnp.broadcast_to(g1, (N_TOK, 16))
    g1_ref[...] = jnp.broadcast_to(g2, (N_TOK, 16))

    g_used = jnp.sum(jnp.where(mask8, nb, 0), axis=1, keepdims=True)
    gi = jax.lax.broadcasted_iota(jnp.int32, (16, 128), 0)
    lane16 = jax.lax.broadcasted_iota(jnp.int32, (16, 128), 1)
    v8 = lane16 < NUM_EXPERTS
    cumb = jnp.broadcast_to(cum_nb, (16, 128))
    be = jnp.sum(jnp.where(v8 & (cumb <= gi), 1, 0), axis=1, keepdims=True)
    be_last = jnp.sum(
        jnp.where(v8 & (cumb <= (g_used[:1, :1] - 1)), 1, 0),
        axis=1, keepdims=True)
    bv = (gi[:, :1] < g_used[:1, :1]).astype(jnp.int32)
    be_ref[...] = jnp.where(bv == 1, be, be_last)
    bv_ref[...] = bv


def _gate_route(x, wg_pad):
    outs = pl.pallas_call(
        _gate_body,
        out_shape=(
            jax.ShapeDtypeStruct((N_TOK, 1), jnp.int32),
            jax.ShapeDtypeStruct((N_TOK, 1), jnp.int32),
            jax.ShapeDtypeStruct((N_TOK, 16), jnp.float32),
            jax.ShapeDtypeStruct((N_TOK, 16), jnp.float32),
            jax.ShapeDtypeStruct((16, 1), jnp.int32),
            jax.ShapeDtypeStruct((16, 1), jnp.int32),
            jax.ShapeDtypeStruct((1, 1), jnp.float32),
        ),
    )(x, wg_pad)
    d0, d1, g0x, g1x, be16, bv16, loss = outs
    return (d0.reshape(N_TOK), d1.reshape(N_TOK), g0x, g1x,
            be16[:GMAX, 0], bv16[:GMAX, 0], loss[0, 0])


def _dispatch_body(x_hbm, d0_hbm, d1_hbm, xs_hbm, xv, i0v, i1v, sem):
    wid = lax.axis_index("s") * 2 + lax.axis_index("c")
    base = wid * TPW
    pltpu.sync_copy(x_hbm.at[pl.ds(base, TPW)], xv)
    pltpu.sync_copy(d0_hbm.at[pl.ds(base, TPW)], i0v)
    pltpu.sync_copy(d1_hbm.at[pl.ds(base, TPW)], i1v)
    pltpu.async_copy(xv, xs_hbm.at[i0v], sem).wait()
    pltpu.async_copy(xv, xs_hbm.at[i1v], sem).wait()


def _dispatch(x, d0, d1):
    return pl.kernel(
        _dispatch_body,
        out_type=jax.ShapeDtypeStruct((PPAD, D_IN), jnp.float32),
        mesh=plsc.VectorSubcoreMesh(core_axis_name="c", subcore_axis_name="s"),
        scratch_types=[pltpu.VMEM((TPW, D_IN), jnp.float32),
                       pltpu.VMEM((TPW,), jnp.int32),
                       pltpu.VMEM((TPW,), jnp.int32),
                       pltpu.SemaphoreType.DMA],
    )(x, d0, d1)


def _combine_body(o_hbm, d0_hbm, d1_hbm, g0_hbm, g1_hbm, y_hbm,
                  i0v, i1v, g0v, g1v, b0, b1, yv, sem):
    wid = lax.axis_index("s") * 2 + lax.axis_index("c")
    for c in range(TPW // CC):
        base = wid * TPW + c * CC
        pltpu.sync_copy(d0_hbm.at[pl.ds(base, CC)], i0v)
        pltpu.sync_copy(d1_hbm.at[pl.ds(base, CC)], i1v)
        pltpu.sync_copy(g0_hbm.at[pl.ds(base, CC)], g0v)
        pltpu.sync_copy(g1_hbm.at[pl.ds(base, CC)], g1v)
        pltpu.async_copy(o_hbm.at[i0v], b0, sem).wait()
        pltpu.async_copy(o_hbm.at[i1v], b1, sem).wait()

        def tok(i, carry):
            ga = g0v[i, :]
            gb = g1v[i, :]
            for j in range(D_OUT // 16):
                sl = pl.ds(j * 16, 16)
                yv[i, sl] = ga * b0[i, sl] + gb * b1[i, sl]
            return carry

        lax.fori_loop(0, CC, tok, 0)
        pltpu.sync_copy(yv, y_hbm.at[pl.ds(base, CC)])


def _combine(o_sorted, d0, d1, g0, g1):
    return pl.kernel(
        _combine_body,
        out_type=jax.ShapeDtypeStruct((N_TOK, D_OUT), jnp.float32),
        mesh=plsc.VectorSubcoreMesh(core_axis_name="c", subcore_axis_name="s"),
        scratch_types=[pltpu.VMEM((CC,), jnp.int32),
                       pltpu.VMEM((CC,), jnp.int32),
                       pltpu.VMEM((CC, 16), jnp.float32),
                       pltpu.VMEM((CC, 16), jnp.float32),
                       pltpu.VMEM((CC, D_OUT), jnp.float32),
                       pltpu.VMEM((CC, D_OUT), jnp.float32),
                       pltpu.VMEM((CC, D_OUT), jnp.float32),
                       pltpu.SemaphoreType.DMA],
    )(o_sorted, d0, d1, g0, g1)


def _ffn_body(be_ref, bv_ref, xb_ref, w1_ref, b1_ref, w2_ref,
              b2_ref, out_ref, acc_ref, xbf_ref):
    g = pl.program_id(0)
    h = pl.program_id(1)

    @pl.when(bv_ref[g] == 1)
    def _():
        @pl.when(h == 0)
        def _():
            xbf_ref[...] = xb_ref[...].astype(jnp.bfloat16)

        hb = jnp.dot(xbf_ref[...], w1_ref[0].astype(jnp.bfloat16),
                     preferred_element_type=jnp.float32)
        hb = jnp.maximum(hb + b1_ref[0], 0.0).astype(jnp.bfloat16)
        contrib = jnp.dot(hb, w2_ref[0].astype(jnp.bfloat16),
                          preferred_element_type=jnp.float32)

        @pl.when(h == 0)
        def _():
            acc_ref[...] = contrib

        @pl.when(h > 0)
        def _():
            acc_ref[...] += contrib

        @pl.when(h == NH - 1)
        def _():
            logits = acc_ref[...] + b2_ref[0]
            m = jnp.max(logits, axis=1, keepdims=True)
            ex = jnp.exp(logits - m)
            out_ref[...] = ex / jnp.sum(ex, axis=1, keepdims=True)


@functools.partial(jax.jit, static_argnames=())
def _ffn(x_sorted, W1, b1, W2, b2, be, bv):
    grid_spec = pltpu.PrefetchScalarGridSpec(
        num_scalar_prefetch=2,
        grid=(GMAX, NH),
        in_specs=[
            pl.BlockSpec((BLK, D_IN), lambda g, h, be, bv: (g, 0)),
            pl.BlockSpec((1, D_IN, HB), lambda g, h, be, bv: (be[g], 0, h)),
            pl.BlockSpec((1, 1, HB), lambda g, h, be, bv: (be[g] * NH + h, 0, 0)),
            pl.BlockSpec((1, HB, D_OUT), lambda g, h, be, bv: (be[g], h, 0)),
            pl.BlockSpec((1, 1, D_OUT), lambda g, h, be, bv: (be[g], 0, 0)),
        ],
        out_specs=pl.BlockSpec((BLK, D_OUT), lambda g, h, be, bv: (g, 0)),
        scratch_shapes=[pltpu.VMEM((BLK, D_OUT), jnp.float32),
                        pltpu.VMEM((BLK, D_IN), jnp.bfloat16)],
    )
    return pl.pallas_call(
        _ffn_body,
        grid_spec=grid_spec,
        out_shape=jax.ShapeDtypeStruct((PPAD, D_OUT), jnp.float32),
        compiler_params=pltpu.CompilerParams(
            dimension_semantics=("arbitrary", "arbitrary")),
    )(be, bv, x_sorted, W1,
      b1.reshape(NUM_EXPERTS * NH, 1, HB), W2,
      b2.reshape(NUM_EXPERTS, 1, D_OUT))


def kernel(x, w_gate, w_noise, W1, b1, W2, b2):
    wg_pad = jnp.pad(w_gate, ((0, 0), (0, 128 - NUM_EXPERTS)))
    d0, d1, g0x, g1x, be, bv, loss = _gate_route(x, wg_pad)
    x_sorted = _dispatch(x, d0, d1)
    o_sorted = _ffn(x_sorted, W1, b1, W2, b2, be, bv)
    y = _combine(o_sorted, d0, d1, g0x, g1x)
    return y, loss
